# baseline (speedup 1.0000x reference)
import functools

import jax
import jax.numpy as jnp
from jax import lax
from jax.experimental import pallas as pl
from jax.experimental.pallas import tpu as pltpu
from jax.experimental.pallas import tpu_sc as plsc

B = 1024
D = 128
E = 128
S = 16
NC = 2
NS = 16
NW = NC * NS
BPW = B // NW
BB = 64
LANES = 16


def _sc_gather_mean(features, nodes, neigh_t, off, nb):
    bpw = nb // NW
    mesh = plsc.VectorSubcoreMesh(core_axis_name="c", subcore_axis_name="s")

    @functools.partial(
        pl.kernel,
        mesh=mesh,
        out_type=[
            jax.ShapeDtypeStruct((nb, D), jnp.float32),
            jax.ShapeDtypeStruct((nb, D), jnp.float32),
        ],
        scratch_types=[
            pltpu.VMEM((bpw,), jnp.int32),
            pltpu.VMEM((S, bpw), jnp.int32),
            pltpu.VMEM((bpw, D), jnp.float32),
            pltpu.VMEM((bpw, D), jnp.float32),
            pltpu.SemaphoreType.DMA,
            pltpu.SemaphoreType.DMA,
        ],
    )
    def k(feat_hbm, nodes_hbm, neigh_hbm, self_out, mean_out,
          sidx, nidx, srows, acc, sem, isem):
        wid = lax.axis_index("s") * NC + lax.axis_index("c")
        base = wid * bpw

        zeros = jnp.zeros((LANES,), jnp.float32)
        for j in range(bpw):
            for c in range(D // LANES):
                acc[j, pl.ds(c * LANES, LANES)] = zeros

        pltpu.sync_copy(nodes_hbm.at[pl.ds(off + base, bpw)], sidx)
        idx_copies = [
            pltpu.async_copy(neigh_hbm.at[pl.ds(s * B + off + base, bpw)],
                             nidx.at[s], isem)
            for s in range(S)
        ]
        for c in idx_copies:
            c.wait()

        copies = [pltpu.async_copy(feat_hbm.at[sidx], srows, sem)]
        for s in range(S):
            copies.append(
                pltpu.async_copy(feat_hbm.at[nidx.at[s]], acc, sem, add=True)
            )
        for c in copies:
            c.wait()

        inv_s = jnp.float32(1.0 / S)

        def body(j, carry):
            for c in range(D // LANES):
                sl = pl.ds(c * LANES, LANES)
                acc[j, sl] = acc[j, sl] * inv_s
            return carry

        lax.fori_loop(0, bpw, body, 0)

        pltpu.sync_copy(srows, self_out.at[pl.ds(base, bpw)])
        pltpu.sync_copy(acc, mean_out.at[pl.ds(base, bpw)])

    return k(features, nodes, neigh_t)


SB = BB // 4


def _tc_encode(self_f, mean_f, weight, mask, off, nb):
    grid = nb // BB
    woff = off // SB
    Ks = SB * 2 * D

    def body(s_ref, m_ref, w0, w1, w2, w3, mask_ref, o_ref):
        comb = jnp.concatenate([s_ref[...], m_ref[...]], axis=1)
        outs = []
        for q, wr in enumerate((w0, w1, w2, w3)):
            cq = comb[q * SB:(q + 1) * SB]
            cdiag = jnp.tile(cq, (1, SB)) * mask_ref[...]
            w = wr[...].reshape(Ks, E)
            outs.append(
                lax.dot_general(cdiag, w, (((1,), (0,)), ((), ())),
                                preferred_element_type=jnp.float32))
        out = jnp.concatenate(outs, axis=0)
        o_ref[...] = jnp.maximum(out, jnp.float32(0.0))

    wspecs = [
        pl.BlockSpec((SB, 2 * D, E), lambda i, q=q: (woff + 4 * i + q, 0, 0))
        for q in range(4)
    ]
    return pl.pallas_call(
        body,
        grid=(grid,),
        in_specs=[
            pl.BlockSpec((BB, D), lambda i: (i, 0)),
            pl.BlockSpec((BB, D), lambda i: (i, 0)),
            *wspecs,
            pl.BlockSpec((SB, Ks), lambda i: (0, 0)),
        ],
        out_specs=pl.BlockSpec((BB, E), lambda i: (i, 0)),
        out_shape=jax.ShapeDtypeStruct((nb, E), jnp.float32),
        compiler_params=pltpu.CompilerParams(
            dimension_semantics=("parallel",)),
    )(self_f, mean_f, weight, weight, weight, weight, mask)


def kernel(features, nodes, neigh_idx, weight):
    nodes = nodes.astype(jnp.int32)
    neigh_t = neigh_idx.astype(jnp.int32).T.reshape(-1)
    Ks = SB * 2 * D
    row = lax.broadcasted_iota(jnp.int32, (SB, Ks), 0)
    grp = lax.broadcasted_iota(jnp.int32, (SB, Ks), 1) // (2 * D)
    mask = (row == grp).astype(jnp.float32)
    nchunks = 2
    nb = B // nchunks
    parts = [_sc_gather_mean(features, nodes, neigh_t, c * nb, nb)
             for c in range(nchunks)]
    outs = [_tc_encode(s, m, weight, mask, c * nb, nb)
            for c, (s, m) in enumerate(parts)]
    return jnp.concatenate(outs, axis=0)


# speedup vs baseline: 1.0167x; 1.0167x over previous
import functools

import jax
import jax.numpy as jnp
from jax import lax
from jax.experimental import pallas as pl
from jax.experimental.pallas import tpu as pltpu
from jax.experimental.pallas import tpu_sc as plsc

B = 1024
D = 128
E = 128
S = 16
NC = 2
NS = 16
NW = NC * NS
BPW = B // NW
BB = 128
LANES = 16


def _sc_gather_mean(features, nodes, neigh_t):
    mesh = plsc.VectorSubcoreMesh(core_axis_name="c", subcore_axis_name="s")

    @functools.partial(
        pl.kernel,
        mesh=mesh,
        out_type=[
            jax.ShapeDtypeStruct((B, D), jnp.float32),
            jax.ShapeDtypeStruct((B, D), jnp.float32),
        ],
        scratch_types=[
            pltpu.VMEM((BPW,), jnp.int32),
            pltpu.VMEM((S, BPW), jnp.int32),
            pltpu.VMEM((BPW, D), jnp.float32),
            pltpu.VMEM((BPW, D), jnp.float32),
            pltpu.SemaphoreType.DMA,
            pltpu.SemaphoreType.DMA,
        ],
    )
    def k(feat_hbm, nodes_hbm, neigh_hbm, self_out, mean_out,
          sidx, nidx, srows, acc, sem, isem):
        wid = lax.axis_index("s") * NC + lax.axis_index("c")
        base = wid * BPW

        zeros = jnp.zeros((LANES,), jnp.float32)
        for j in range(BPW):
            for c in range(D // LANES):
                acc[j, pl.ds(c * LANES, LANES)] = zeros

        pltpu.sync_copy(nodes_hbm.at[pl.ds(base, BPW)], sidx)
        idx_copies = [
            pltpu.async_copy(neigh_hbm.at[pl.ds(s * B + base, BPW)], nidx.at[s], isem)
            for s in range(S)
        ]
        for c in idx_copies:
            c.wait()

        copies = [pltpu.async_copy(feat_hbm.at[sidx], srows, sem)]
        for s in range(S):
            copies.append(
                pltpu.async_copy(feat_hbm.at[nidx.at[s]], acc, sem, add=True)
            )
        for c in copies:
            c.wait()

        inv_s = jnp.float32(1.0 / S)

        def body(j, carry):
            for c in range(D // LANES):
                sl = pl.ds(c * LANES, LANES)
                acc[j, sl] = acc[j, sl] * inv_s
            return carry

        lax.fori_loop(0, BPW, body, 0)

        pltpu.sync_copy(srows, self_out.at[pl.ds(base, BPW)])
        pltpu.sync_copy(acc, mean_out.at[pl.ds(base, BPW)])

    return k(features, nodes, neigh_t)


SB = BB // 4


def _tc_encode(self_f, mean_f, weight, mask):
    grid = B // BB
    Ks = SB * 2 * D

    def body(s_ref, m_ref, w0, w1, w2, w3, mask_ref, o_ref):
        comb = jnp.concatenate([s_ref[...], m_ref[...]], axis=1)
        outs = []
        for q, wr in enumerate((w0, w1, w2, w3)):
            cq = comb[q * SB:(q + 1) * SB]
            cdiag = jnp.tile(cq, (1, SB)) * mask_ref[...]
            w = wr[...].reshape(Ks, E)
            outs.append(
                lax.dot_general(cdiag, w, (((1,), (0,)), ((), ())),
                                preferred_element_type=jnp.float32))
        out = jnp.concatenate(outs, axis=0)
        o_ref[...] = jnp.maximum(out, jnp.float32(0.0))

    wspecs = [
        pl.BlockSpec((SB, 2 * D, E), lambda i, q=q: (4 * i + q, 0, 0))
        for q in range(4)
    ]
    return pl.pallas_call(
        body,
        grid=(grid,),
        in_specs=[
            pl.BlockSpec((BB, D), lambda i: (i, 0)),
            pl.BlockSpec((BB, D), lambda i: (i, 0)),
            *wspecs,
            pl.BlockSpec((SB, Ks), lambda i: (0, 0)),
        ],
        out_specs=pl.BlockSpec((BB, E), lambda i: (i, 0)),
        out_shape=jax.ShapeDtypeStruct((B, E), jnp.float32),
        compiler_params=pltpu.CompilerParams(
            dimension_semantics=("parallel",)),
    )(self_f, mean_f, weight, weight, weight, weight, mask)


def kernel(features, nodes, neigh_idx, weight):
    nodes = nodes.astype(jnp.int32)
    neigh_t = neigh_idx.astype(jnp.int32).T.reshape(-1)
    Ks = SB * 2 * D
    row = lax.broadcasted_iota(jnp.int32, (SB, Ks), 0)
    grp = lax.broadcasted_iota(jnp.int32, (SB, Ks), 1) // (2 * D)
    mask = (row == grp).astype(jnp.float32)
    self_f, mean_f = _sc_gather_mean(features, nodes, neigh_t)
    return _tc_encode(self_f, mean_f, weight, mask)


# speedup vs baseline: 1.0611x; 1.0437x over previous
import functools

import jax
import jax.numpy as jnp
from jax import lax
from jax.experimental import pallas as pl
from jax.experimental.pallas import tpu as pltpu
from jax.experimental.pallas import tpu_sc as plsc

B = 1024
D = 128
E = 128
S = 16
NC = 2
NS = 16
NW = NC * NS
BPW = B // NW
BB = 64
LANES = 16


def _sc_gather_mean(features, nodes, neigh_t):
    mesh = plsc.VectorSubcoreMesh(core_axis_name="c", subcore_axis_name="s")

    @functools.partial(
        pl.kernel,
        mesh=mesh,
        out_type=[
            jax.ShapeDtypeStruct((B, D), jnp.float32),
            jax.ShapeDtypeStruct((B, D), jnp.float32),
        ],
        scratch_types=[
            pltpu.VMEM((BPW,), jnp.int32),
            pltpu.VMEM((S, BPW), jnp.int32),
            pltpu.VMEM((BPW, D), jnp.float32),
            pltpu.VMEM((BPW, D), jnp.float32),
            pltpu.SemaphoreType.DMA,
            pltpu.SemaphoreType.DMA,
        ],
    )
    def k(feat_hbm, nodes_hbm, neigh_hbm, self_out, mean_out,
          sidx, nidx, srows, acc, sem, isem):
        wid = lax.axis_index("s") * NC + lax.axis_index("c")
        base = wid * BPW

        zeros = jnp.zeros((LANES,), jnp.float32)
        for j in range(BPW):
            for c in range(D // LANES):
                acc[j, pl.ds(c * LANES, LANES)] = zeros

        pltpu.sync_copy(nodes_hbm.at[pl.ds(base, BPW)], sidx)
        idx_copies = [
            pltpu.async_copy(neigh_hbm.at[pl.ds(s * B + base, BPW)], nidx.at[s], isem)
            for s in range(S)
        ]
        for c in idx_copies:
            c.wait()

        copies = [pltpu.async_copy(feat_hbm.at[sidx], srows, sem)]
        for s in range(S):
            copies.append(
                pltpu.async_copy(feat_hbm.at[nidx.at[s]], acc, sem, add=True)
            )
        for c in copies:
            c.wait()

        inv_s = jnp.float32(1.0 / S)

        def body(j, carry):
            for c in range(D // LANES):
                sl = pl.ds(c * LANES, LANES)
                acc[j, sl] = acc[j, sl] * inv_s
            return carry

        lax.fori_loop(0, BPW, body, 0)

        pltpu.sync_copy(srows, self_out.at[pl.ds(base, BPW)])
        pltpu.sync_copy(acc, mean_out.at[pl.ds(base, BPW)])

    return k(features, nodes, neigh_t)


SB = BB // 4


def _tc_encode(self_f, mean_f, weight, mask):
    grid = B // BB
    Ks = SB * 2 * D

    def body(s_ref, m_ref, w0, w1, w2, w3, mask_ref, o_ref):
        comb = jnp.concatenate([s_ref[...], m_ref[...]], axis=1)
        outs = []
        for q, wr in enumerate((w0, w1, w2, w3)):
            cq = comb[q * SB:(q + 1) * SB]
            cdiag = jnp.tile(cq, (1, SB)) * mask_ref[...]
            w = wr[...].reshape(Ks, E)
            outs.append(
                lax.dot_general(cdiag, w, (((1,), (0,)), ((), ())),
                                preferred_element_type=jnp.float32))
        out = jnp.concatenate(outs, axis=0)
        o_ref[...] = jnp.maximum(out, jnp.float32(0.0))

    wspecs = [
        pl.BlockSpec((SB, 2 * D, E), lambda i, q=q: (4 * i + q, 0, 0))
        for q in range(4)
    ]
    return pl.pallas_call(
        body,
        grid=(grid,),
        in_specs=[
            pl.BlockSpec((BB, D), lambda i: (i, 0)),
            pl.BlockSpec((BB, D), lambda i: (i, 0)),
            *wspecs,
            pl.BlockSpec((SB, Ks), lambda i: (0, 0)),
        ],
        out_specs=pl.BlockSpec((BB, E), lambda i: (i, 0)),
        out_shape=jax.ShapeDtypeStruct((B, E), jnp.float32),
        compiler_params=pltpu.CompilerParams(
            dimension_semantics=("parallel",)),
    )(self_f, mean_f, weight, weight, weight, weight, mask)


def kernel(features, nodes, neigh_idx, weight):
    nodes = nodes.astype(jnp.int32)
    neigh_t = neigh_idx.astype(jnp.int32).T.reshape(-1)
    Ks = SB * 2 * D
    row = lax.broadcasted_iota(jnp.int32, (SB, Ks), 0)
    grp = lax.broadcasted_iota(jnp.int32, (SB, Ks), 1) // (2 * D)
    mask = (row == grp).astype(jnp.float32)
    self_f, mean_f = _sc_gather_mean(features, nodes, neigh_t)
    return _tc_encode(self_f, mean_f, weight, mask)
